# Initial kernel scaffold; baseline (speedup 1.0000x reference)
import jax
import jax.numpy as jnp
from jax.experimental import pallas as pl


def kernel(emb_table, idx, targets):
    raise NotImplementedError("write your pallas kernel here")


# speedup vs baseline: 1.3239x; 1.3239x over previous
import dataclasses
import functools

import jax
import jax.numpy as jnp
from jax import lax
from jax.experimental import pallas as pl
from jax.experimental.pallas import tpu as pltpu
from jax.experimental.pallas import tpu_sc as plsc

V = 1000
C = 1000
B = 1024
T = 200
N = B * T
NC = 2
NS = 16
NW = NC * NS
L = 16
W = 32
VPAD = 1024


def _row_lse_body(tab_ref, out_ref):
    x = tab_ref[...]
    m = jnp.max(x, axis=1, keepdims=True)
    s = jnp.sum(jnp.exp(x - m), axis=1, keepdims=True)
    out_ref[...] = jnp.log(s) + m


def _loss_body(part_ref, out_ref):
    out_ref[...] = (jnp.sum(part_ref[...]) / N).reshape(1, 1)


def _sc_body(table_hbm, idx_hbm, tgt_hbm, lse_hbm, out_hbm, part_hbm,
             idx_v, tgt_v, lse_v, acc_v, buf0, buf1, gsem, psem):
    wid = lax.axis_index(("c", "s"))
    npw = N // NW
    steps = npw // W
    base = wid * npw

    pltpu.sync_copy(idx_hbm.at[pl.ds(base, npw)], idx_v)
    pltpu.sync_copy(tgt_hbm.at[pl.ds(base, npw)], tgt_v)
    pltpu.sync_copy(lse_hbm, lse_v)
    acc_v[...] = jnp.zeros((L,), jnp.float32)

    def start_gather(s, buf):
        pltpu.async_copy(table_hbm.at[idx_v.at[pl.ds(s * W, W)]], buf, gsem)

    def wait_gather(buf):
        pltpu.make_async_copy(table_hbm.at[idx_v.at[pl.ds(0, W)]], buf,
                              gsem).wait()

    def start_put(s, buf):
        pltpu.async_copy(buf, out_hbm.at[pl.ds(base + s * W, W)], psem)

    def wait_put():
        pltpu.make_async_copy(buf0, out_hbm.at[pl.ds(base, W)], psem).wait()

    def compute(s, buf):
        for g in range(W // L):
            rows = lax.iota(jnp.int32, L) + g * L
            cols = tgt_v[pl.ds(s * W + g * L, L)]
            ivals = idx_v[pl.ds(s * W + g * L, L)]
            picked = plsc.load_gather(buf, [rows, cols])
            lses = plsc.load_gather(lse_v, [ivals])
            acc_v[...] += lses - picked

    def stage(s, buf, other):
        wait_gather(buf)

        @pl.when(s >= 1)
        def _():
            wait_put()

        @pl.when(s + 1 < steps)
        def _():
            start_gather(s + 1, other)

        compute(s, buf)
        start_put(s, buf)

    start_gather(0, buf0)

    @pl.loop(0, steps, step=2)
    def _(s):
        stage(s, buf0, buf1)
        stage(s + 1, buf1, buf0)

    wait_put()
    pltpu.sync_copy(acc_v, part_hbm.at[wid])


@jax.jit
def kernel(emb_table, idx, targets):
    row_lse = pl.pallas_call(
        _row_lse_body,
        out_shape=jax.ShapeDtypeStruct((V, 1), jnp.float32),
    )(emb_table)
    lse_pad = jnp.zeros((VPAD,), jnp.float32).at[:V].set(row_lse[:, 0])

    idx_flat = idx.astype(jnp.int32).reshape(N)
    tgt_flat = targets.astype(jnp.int32).reshape(N)

    mesh = plsc.VectorSubcoreMesh(core_axis_name="c", subcore_axis_name="s")
    cp = dataclasses.replace(pltpu.CompilerParams(),
                             needs_layout_passes=False,
                             use_tc_tiling_on_sc=False)
    sc_gather = pl.kernel(
        _sc_body,
        out_type=[
            jax.ShapeDtypeStruct((N, C), jnp.float32),
            jax.ShapeDtypeStruct((NW, L), jnp.float32),
        ],
        mesh=mesh,
        compiler_params=cp,
        scratch_types=[
            pltpu.VMEM((N // NW,), jnp.int32),
            pltpu.VMEM((N // NW,), jnp.int32),
            pltpu.VMEM((VPAD,), jnp.float32),
            pltpu.VMEM((L,), jnp.float32),
            pltpu.VMEM((W, C), jnp.float32),
            pltpu.VMEM((W, C), jnp.float32),
            pltpu.SemaphoreType.DMA,
            pltpu.SemaphoreType.DMA,
        ],
    )
    logits_flat, partials = sc_gather(emb_table, idx_flat, tgt_flat, lse_pad)

    loss = pl.pallas_call(
        _loss_body,
        out_shape=jax.ShapeDtypeStruct((1, 1), jnp.float32),
    )(partials)[0, 0]

    return logits_flat.reshape(B, T, C), loss


# speedup vs baseline: 2.0287x; 1.5324x over previous
import dataclasses
import functools

import jax
import jax.numpy as jnp
from jax import lax
from jax.experimental import pallas as pl
from jax.experimental.pallas import tpu as pltpu
from jax.experimental.pallas import tpu_sc as plsc

V = 1000
C = 1000
CP = 1024
B = 1024
T = 200
N = B * T
NC = 2
NS = 16
NW = NC * NS
L = 16
W = 32
KB = 640


def _row_lse_body(tab_ref, out_ref):
    x = tab_ref[...]
    m = jnp.max(x, axis=1, keepdims=True)
    s = jnp.sum(jnp.exp(x - m), axis=1, keepdims=True)
    out_ref[...] = jnp.log(s) + m


def _loss_body(part_ref, out_ref):
    out_ref[...] = (jnp.sum(part_ref[...]) / N).reshape(1, 1)


def _sc_gather_body(table_hbm, idx_hbm, out_hbm, idx_v, buf0, buf1,
                    gsem, psem):
    wid = lax.axis_index(("c", "s"))
    npw = N // NW
    steps = npw // W
    base = wid * npw

    pltpu.sync_copy(idx_hbm.at[pl.ds(base, npw)], idx_v)

    def start_gather(s, buf):
        pltpu.async_copy(table_hbm.at[idx_v.at[pl.ds(s * W, W)]], buf, gsem)

    def wait_gather(buf):
        pltpu.make_async_copy(table_hbm.at[idx_v.at[pl.ds(0, W)]], buf,
                              gsem).wait()

    def start_put(s, buf):
        pltpu.async_copy(buf, out_hbm.at[pl.ds(base + s * W, W)], psem)

    def wait_put():
        pltpu.make_async_copy(buf0, out_hbm.at[pl.ds(base, W)], psem).wait()

    def stage(s, buf, other):
        wait_gather(buf)

        @pl.when(s >= 1)
        def _():
            wait_put()

        @pl.when(s + 1 < steps)
        def _():
            start_gather(s + 1, other)

        start_put(s, buf)

    start_gather(0, buf0)

    @pl.loop(0, steps, step=2)
    def _(s):
        stage(s, buf0, buf1)
        stage(s + 1, buf1, buf0)

    wait_put()


def _sc_loss_body(tabf_hbm, lse_hbm, fidx_hbm, idx_hbm, part_hbm,
                  pick_v, lse_v, acc_v, sem):
    wid = lax.axis_index(("c", "s"))
    npw = N // NW
    chunks = npw // KB
    base = wid * npw

    acc_v[...] = jnp.zeros((L,), jnp.float32)

    def run_scoped_body(fidx_v, iidx_v):
        @pl.loop(0, chunks)
        def _(k):
            pltpu.sync_copy(fidx_hbm.at[pl.ds(base + k * KB, KB)], fidx_v)
            pltpu.sync_copy(idx_hbm.at[pl.ds(base + k * KB, KB)], iidx_v)
            pltpu.async_copy(tabf_hbm.at[fidx_v], pick_v, sem).wait()
            pltpu.async_copy(lse_hbm.at[iidx_v], lse_v, sem).wait()
            for g in range(KB // L):
                acc_v[...] += (lse_v[pl.ds(g * L, L)]
                               - pick_v[pl.ds(g * L, L)])

    pl.run_scoped(run_scoped_body,
                  pltpu.VMEM((KB,), jnp.int32),
                  pltpu.VMEM((KB,), jnp.int32))
    pltpu.sync_copy(acc_v, part_hbm.at[wid])


@jax.jit
def kernel(emb_table, idx, targets):
    row_lse = pl.pallas_call(
        _row_lse_body,
        out_shape=jax.ShapeDtypeStruct((V, 1), jnp.float32),
    )(emb_table)
    lse_flat = jnp.zeros((B,), jnp.float32).at[:V].set(row_lse[:, 0])

    table_pad = jnp.pad(emb_table, ((0, 0), (0, CP - C)))
    idx_flat = idx.astype(jnp.int32).reshape(N)
    tgt_flat = targets.astype(jnp.int32).reshape(N)
    fidx = idx_flat * CP + tgt_flat

    mesh = plsc.VectorSubcoreMesh(core_axis_name="c", subcore_axis_name="s")
    cp_gather = dataclasses.replace(pltpu.CompilerParams(),
                                    needs_layout_passes=False)
    cp_loss = dataclasses.replace(pltpu.CompilerParams(),
                                  needs_layout_passes=False,
                                  use_tc_tiling_on_sc=False)

    sc_gather = pl.kernel(
        _sc_gather_body,
        out_type=jax.ShapeDtypeStruct((N, CP), jnp.float32),
        mesh=mesh,
        compiler_params=cp_gather,
        scratch_types=[
            pltpu.VMEM((N // NW,), jnp.int32),
            pltpu.VMEM((W, CP), jnp.float32),
            pltpu.VMEM((W, CP), jnp.float32),
            pltpu.SemaphoreType.DMA,
            pltpu.SemaphoreType.DMA,
        ],
    )
    logits_flat = sc_gather(table_pad, idx_flat)[:, :C]

    sc_loss = pl.kernel(
        _sc_loss_body,
        out_type=jax.ShapeDtypeStruct((NW, L), jnp.float32),
        mesh=mesh,
        compiler_params=cp_loss,
        scratch_types=[
            pltpu.VMEM((KB,), jnp.float32),
            pltpu.VMEM((KB,), jnp.float32),
            pltpu.VMEM((L,), jnp.float32),
            pltpu.SemaphoreType.DMA,
        ],
    )
    partials = sc_loss(table_pad.reshape(V * CP), lse_flat, fidx, idx_flat)

    loss = pl.pallas_call(
        _loss_body,
        out_shape=jax.ShapeDtypeStruct((1, 1), jnp.float32),
    )(partials)[0, 0]

    return logits_flat.reshape(B, T, C), loss


# speedup vs baseline: 2.1147x; 1.0424x over previous
import dataclasses
import functools

import jax
import jax.numpy as jnp
from jax import lax
from jax.experimental import pallas as pl
from jax.experimental.pallas import tpu as pltpu
from jax.experimental.pallas import tpu_sc as plsc

V = 1000
C = 1000
CP = 1024
B = 1024
T = 200
N = B * T
NC = 2
NS = 16
NW = NC * NS
L = 16
NB = 8
TCK = T // NB
RPC = TCK * B
W = 40
KB = 640


def _row_lse_body(tab_ref, out_ref):
    x = tab_ref[...]
    m = jnp.max(x, axis=1, keepdims=True)
    s = jnp.sum(jnp.exp(x - m), axis=1, keepdims=True)
    out_ref[...] = jnp.log(s) + m


def _loss_body(part_ref, out_ref):
    out_ref[...] = (jnp.sum(part_ref[...]) / N).reshape(1, 1)


def _transpose_body(carry_ref, in_ref, out_ref):
    del carry_ref
    out_ref[0] = jnp.transpose(in_ref[...])[:C, :]


def _transpose_body0(in_ref, out_ref):
    out_ref[0] = jnp.transpose(in_ref[...])[:C, :]


def _sc_gather_body(table_hbm, idx_hbm, out_hbm, idx_v, buf0, buf1,
                    gsem, psem):
    wid = lax.axis_index(("c", "s"))
    npw = RPC // NW
    steps = npw // W
    base = wid * npw

    pltpu.sync_copy(idx_hbm.at[pl.ds(base, npw)], idx_v)

    def start_gather(s, buf):
        pltpu.async_copy(table_hbm.at[idx_v.at[pl.ds(s * W, W)]], buf, gsem)

    def wait_gather(buf):
        pltpu.make_async_copy(table_hbm.at[idx_v.at[pl.ds(0, W)]], buf,
                              gsem).wait()

    def start_put(s, buf):
        pltpu.async_copy(buf, out_hbm.at[pl.ds(base + s * W, W)], psem)

    def wait_put():
        pltpu.make_async_copy(buf0, out_hbm.at[pl.ds(base, W)], psem).wait()

    def stage(s, buf, other):
        wait_gather(buf)

        @pl.when(s >= 1)
        def _():
            wait_put()

        @pl.when(s + 1 < steps)
        def _():
            start_gather(s + 1, other)

        start_put(s, buf)

    start_gather(0, buf0)

    @pl.loop(0, steps, step=2)
    def _(s):
        stage(s, buf0, buf1)
        stage(s + 1, buf1, buf0)

    wait_put()


def _sc_loss_body(tabf_hbm, lse_hbm, fidx_hbm, idx_hbm, part_hbm,
                  pick_v, lse_v, acc_v, sem):
    wid = lax.axis_index(("c", "s"))
    npw = N // NW
    chunks = npw // KB
    base = wid * npw

    acc_v[...] = jnp.zeros((L,), jnp.float32)

    def run_scoped_body(fidx_v, iidx_v):
        @pl.loop(0, chunks)
        def _(k):
            pltpu.sync_copy(fidx_hbm.at[pl.ds(base + k * KB, KB)], fidx_v)
            pltpu.sync_copy(idx_hbm.at[pl.ds(base + k * KB, KB)], iidx_v)
            pltpu.async_copy(tabf_hbm.at[fidx_v], pick_v, sem).wait()
            pltpu.async_copy(lse_hbm.at[iidx_v], lse_v, sem).wait()
            for g in range(KB // L):
                acc_v[...] += (lse_v[pl.ds(g * L, L)]
                               - pick_v[pl.ds(g * L, L)])

    pl.run_scoped(run_scoped_body,
                  pltpu.VMEM((KB,), jnp.int32),
                  pltpu.VMEM((KB,), jnp.int32))
    pltpu.sync_copy(acc_v, part_hbm.at[wid])


@jax.jit
def kernel(emb_table, idx, targets):
    row_lse = pl.pallas_call(
        _row_lse_body,
        out_shape=jax.ShapeDtypeStruct((V, 1), jnp.float32),
    )(emb_table)
    lse_flat = jnp.zeros((B,), jnp.float32).at[:V].set(row_lse[:, 0])

    table_pad = jnp.pad(emb_table, ((0, 0), (0, CP - C)))
    idx_flat = idx.astype(jnp.int32).reshape(N)
    tgt_flat = targets.astype(jnp.int32).reshape(N)
    fidx = idx_flat * CP + tgt_flat
    idx_t = jnp.transpose(idx.astype(jnp.int32)).reshape(N)

    mesh = plsc.VectorSubcoreMesh(core_axis_name="c", subcore_axis_name="s")
    cp_gather = dataclasses.replace(pltpu.CompilerParams(),
                                    needs_layout_passes=False)
    cp_loss = dataclasses.replace(pltpu.CompilerParams(),
                                  needs_layout_passes=False,
                                  use_tc_tiling_on_sc=False)

    sc_gather = pl.kernel(
        _sc_gather_body,
        out_type=jax.ShapeDtypeStruct((RPC, CP), jnp.float32),
        mesh=mesh,
        compiler_params=cp_gather,
        scratch_types=[
            pltpu.VMEM((RPC // NW,), jnp.int32),
            pltpu.VMEM((W, CP), jnp.float32),
            pltpu.VMEM((W, CP), jnp.float32),
            pltpu.SemaphoreType.DMA,
            pltpu.SemaphoreType.DMA,
        ],
    )
    chunks = [sc_gather(table_pad, idx_t[k * RPC:(k + 1) * RPC])
              for k in range(NB)]

    sc_loss = pl.kernel(
        _sc_loss_body,
        out_type=jax.ShapeDtypeStruct((NW, L), jnp.float32),
        mesh=mesh,
        compiler_params=cp_loss,
        scratch_types=[
            pltpu.VMEM((KB,), jnp.float32),
            pltpu.VMEM((KB,), jnp.float32),
            pltpu.VMEM((L,), jnp.float32),
            pltpu.SemaphoreType.DMA,
        ],
    )
    partials = sc_loss(table_pad.reshape(V * CP), lse_flat, fidx, idx_flat)

    def transpose_chunk(k, carry, chunk):
        return pl.pallas_call(
            _transpose_body,
            grid=(TCK,),
            in_specs=[
                pl.BlockSpec(memory_space=pl.ANY),
                pl.BlockSpec((B, CP), lambda i: (i, 0)),
            ],
            out_specs=pl.BlockSpec((1, C, B), lambda i, k=k: (k * TCK + i,
                                                              0, 0)),
            out_shape=jax.ShapeDtypeStruct((T, C, B), jnp.float32),
            input_output_aliases={0: 0},
        )(carry, chunk)

    acc = pl.pallas_call(
        _transpose_body0,
        grid=(TCK,),
        in_specs=[pl.BlockSpec((B, CP), lambda i: (i, 0))],
        out_specs=pl.BlockSpec((1, C, B), lambda i: (i, 0, 0)),
        out_shape=jax.ShapeDtypeStruct((T, C, B), jnp.float32),
    )(chunks[0])
    for k in range(1, NB):
        acc = transpose_chunk(k, acc, chunks[k])

    loss = pl.pallas_call(
        _loss_body,
        out_shape=jax.ShapeDtypeStruct((1, 1), jnp.float32),
    )(partials)[0, 0]

    return jnp.transpose(acc, (2, 0, 1)), loss


# speedup vs baseline: 2.1245x; 1.0046x over previous
import dataclasses
import functools

import jax
import jax.numpy as jnp
from jax import lax
from jax.experimental import pallas as pl
from jax.experimental.pallas import tpu as pltpu
from jax.experimental.pallas import tpu_sc as plsc

V = 1000
C = 1000
CP = 1024
B = 1024
T = 200
N = B * T
NC = 2
NS = 16
NW = NC * NS
L = 16
NB = 8
TCK = T // NB
RPC = TCK * B
W = 40
KB = 640


def _row_lse_body(tab_ref, out_ref):
    x = tab_ref[...]
    m = jnp.max(x, axis=1, keepdims=True)
    s = jnp.sum(jnp.exp(x - m), axis=1, keepdims=True)
    out_ref[...] = jnp.log(s) + m


def _loss_body(part_ref, out_ref):
    out_ref[...] = (jnp.sum(part_ref[...]) / N).reshape(1, 1)


def _transpose_strips(in_ref, out_ref):
    for j in range(4):
        out_ref[0, :, pl.ds(j * 256, 256)] = (
            jnp.transpose(in_ref[pl.ds(j * 256, 256), :])[:C, :])


def _transpose_body(carry_ref, in_ref, out_ref):
    del carry_ref
    _transpose_strips(in_ref, out_ref)


def _transpose_body0(in_ref, out_ref):
    _transpose_strips(in_ref, out_ref)


def _sc_gather_body(table_hbm, idx_hbm, out_hbm, idx_v, buf0, buf1,
                    gsem, psem):
    wid = lax.axis_index(("c", "s"))
    npw = RPC // NW
    steps = npw // W
    base = wid * npw

    pltpu.sync_copy(idx_hbm.at[pl.ds(base, npw)], idx_v)

    def start_gather(s, buf):
        pltpu.async_copy(table_hbm.at[idx_v.at[pl.ds(s * W, W)]], buf, gsem)

    def wait_gather(buf):
        pltpu.make_async_copy(table_hbm.at[idx_v.at[pl.ds(0, W)]], buf,
                              gsem).wait()

    def start_put(s, buf):
        pltpu.async_copy(buf, out_hbm.at[pl.ds(base + s * W, W)], psem)

    def wait_put():
        pltpu.make_async_copy(buf0, out_hbm.at[pl.ds(base, W)], psem).wait()

    def stage(s, buf, other):
        wait_gather(buf)

        @pl.when(s >= 1)
        def _():
            wait_put()

        @pl.when(s + 1 < steps)
        def _():
            start_gather(s + 1, other)

        start_put(s, buf)

    start_gather(0, buf0)

    @pl.loop(0, steps, step=2)
    def _(s):
        stage(s, buf0, buf1)
        stage(s + 1, buf1, buf0)

    wait_put()


def _sc_loss_body(tabf_hbm, lse_hbm, fidx_hbm, idx_hbm, part_hbm,
                  pick_v, lse_v, acc_v, sem):
    wid = lax.axis_index(("c", "s"))
    npw = N // NW
    chunks = npw // KB
    base = wid * npw

    acc_v[...] = jnp.zeros((L,), jnp.float32)

    def run_scoped_body(fidx_v, iidx_v):
        @pl.loop(0, chunks)
        def _(k):
            pltpu.sync_copy(fidx_hbm.at[pl.ds(base + k * KB, KB)], fidx_v)
            pltpu.sync_copy(idx_hbm.at[pl.ds(base + k * KB, KB)], iidx_v)
            pltpu.async_copy(tabf_hbm.at[fidx_v], pick_v, sem).wait()
            pltpu.async_copy(lse_hbm.at[iidx_v], lse_v, sem).wait()
            for g in range(KB // L):
                acc_v[...] += (lse_v[pl.ds(g * L, L)]
                               - pick_v[pl.ds(g * L, L)])

    pl.run_scoped(run_scoped_body,
                  pltpu.VMEM((KB,), jnp.int32),
                  pltpu.VMEM((KB,), jnp.int32))
    pltpu.sync_copy(acc_v, part_hbm.at[wid])


@jax.jit
def kernel(emb_table, idx, targets):
    row_lse = pl.pallas_call(
        _row_lse_body,
        out_shape=jax.ShapeDtypeStruct((V, 1), jnp.float32),
    )(emb_table)
    lse_flat = jnp.zeros((B,), jnp.float32).at[:V].set(row_lse[:, 0])

    table_pad = jnp.pad(emb_table, ((0, 0), (0, CP - C)))
    idx_flat = idx.astype(jnp.int32).reshape(N)
    tgt_flat = targets.astype(jnp.int32).reshape(N)
    fidx = idx_flat * CP + tgt_flat
    idx_t = jnp.transpose(idx.astype(jnp.int32)).reshape(N)

    mesh = plsc.VectorSubcoreMesh(core_axis_name="c", subcore_axis_name="s")
    cp_gather = dataclasses.replace(pltpu.CompilerParams(),
                                    needs_layout_passes=False)
    cp_loss = dataclasses.replace(pltpu.CompilerParams(),
                                  needs_layout_passes=False,
                                  use_tc_tiling_on_sc=False)

    sc_gather = pl.kernel(
        _sc_gather_body,
        out_type=jax.ShapeDtypeStruct((RPC, CP), jnp.float32),
        mesh=mesh,
        compiler_params=cp_gather,
        scratch_types=[
            pltpu.VMEM((RPC // NW,), jnp.int32),
            pltpu.VMEM((W, CP), jnp.float32),
            pltpu.VMEM((W, CP), jnp.float32),
            pltpu.SemaphoreType.DMA,
            pltpu.SemaphoreType.DMA,
        ],
    )
    chunks = [sc_gather(table_pad, idx_t[k * RPC:(k + 1) * RPC])
              for k in range(NB)]

    sc_loss = pl.kernel(
        _sc_loss_body,
        out_type=jax.ShapeDtypeStruct((NW, L), jnp.float32),
        mesh=mesh,
        compiler_params=cp_loss,
        scratch_types=[
            pltpu.VMEM((KB,), jnp.float32),
            pltpu.VMEM((KB,), jnp.float32),
            pltpu.VMEM((L,), jnp.float32),
            pltpu.SemaphoreType.DMA,
        ],
    )
    lse_dep = lse_flat.at[V].add(chunks[-1][0, 0] * 0.0)
    partials = sc_loss(table_pad.reshape(V * CP), lse_dep, fidx, idx_flat)

    def transpose_chunk(k, carry, chunk):
        return pl.pallas_call(
            _transpose_body,
            grid=(TCK,),
            in_specs=[
                pl.BlockSpec(memory_space=pl.ANY),
                pl.BlockSpec((B, CP), lambda i: (i, 0)),
            ],
            out_specs=pl.BlockSpec((1, C, B), lambda i, k=k: (k * TCK + i,
                                                              0, 0)),
            out_shape=jax.ShapeDtypeStruct((T, C, B), jnp.float32),
            input_output_aliases={0: 0},
        )(carry, chunk)

    acc = pl.pallas_call(
        _transpose_body0,
        grid=(TCK,),
        in_specs=[pl.BlockSpec((B, CP), lambda i: (i, 0))],
        out_specs=pl.BlockSpec((1, C, B), lambda i: (i, 0, 0)),
        out_shape=jax.ShapeDtypeStruct((T, C, B), jnp.float32),
    )(chunks[0])
    for k in range(1, NB):
        acc = transpose_chunk(k, acc, chunks[k])

    loss = pl.pallas_call(
        _loss_body,
        out_shape=jax.ShapeDtypeStruct((1, 1), jnp.float32),
    )(partials)[0, 0]

    return jnp.transpose(acc, (2, 0, 1)), loss
